# baseline (device time: 799848 ns/iter reference)
import jax
import jax.numpy as jnp
from jax import lax
from jax.experimental import pallas as pl
from jax.experimental.pallas import tpu as pltpu

P = 32
M_PER = 128
N = 8192
K_SH = 128
COMM_DTYPE = jnp.bfloat16


def kernel(x, w_mat, scale_x, scale_w):
    m_total, k_sh = x.shape
    _, n = w_mat.shape
    assert (m_total, k_sh, n) == (P * M_PER, K_SH, N), (x.shape, w_mat.shape)

    def body(x_ref, w_ref, sx_ref, sw_ref, out_ref,
             recv_buf, send_buf, send_sems, recv_sems, credit_sem):
        my = lax.axis_index("i")
        left = lax.rem(my + P - 1, P)
        right = lax.rem(my + 1, P)

        barrier = pltpu.get_barrier_semaphore()
        for nbr in (left, right):
            pl.semaphore_signal(barrier, inc=1, device_id=(nbr,),
                                device_id_type=pl.DeviceIdType.MESH)
        pl.semaphore_wait(barrier, 2)

        def partial_chunk(c):
            xa = x_ref[pl.ds(c * M_PER, M_PER), :]
            return lax.dot_general(
                xa, w_ref[:, :], (((1,), (0,)), ((), ())),
                preferred_element_type=jnp.float32)

        def hop_rdma(h):
            slot = h % 2
            return pltpu.make_async_remote_copy(
                src_ref=send_buf.at[slot],
                dst_ref=recv_buf.at[slot],
                send_sem=send_sems.at[slot],
                recv_sem=recv_sems.at[slot],
                device_id=(right,),
                device_id_type=pl.DeviceIdType.MESH)

        for h in range(P - 1):
            c = lax.rem(my + (P - 1 - h), P)
            part = partial_chunk(c)
            if h > 0:
                hop_rdma(h - 1).wait_recv()
                part = part + recv_buf[(h - 1) % 2].astype(jnp.float32)
                if h - 1 <= P - 4:
                    pl.semaphore_signal(credit_sem, inc=1, device_id=(left,),
                                        device_id_type=pl.DeviceIdType.MESH)
            send_buf[h % 2] = part.astype(COMM_DTYPE)
            if h >= 2:
                pl.semaphore_wait(credit_sem, 1)
            r = hop_rdma(h)
            r.start()
            r.wait_send()

        hop_rdma(P - 2).wait_recv()
        acc = partial_chunk(my) + recv_buf[(P - 2) % 2].astype(jnp.float32)
        scale = sx_ref[0] * sw_ref[0]
        out_ref[:, :] = jnp.maximum(acc * scale, 0.0)

    return pl.pallas_call(
        body,
        out_shape=jax.ShapeDtypeStruct((M_PER, N), jnp.float32),
        in_specs=[
            pl.BlockSpec(memory_space=pltpu.VMEM),
            pl.BlockSpec(memory_space=pltpu.VMEM),
            pl.BlockSpec(memory_space=pltpu.SMEM),
            pl.BlockSpec(memory_space=pltpu.SMEM),
        ],
        out_specs=pl.BlockSpec(memory_space=pltpu.VMEM),
        scratch_shapes=[
            pltpu.VMEM((2, M_PER, N), COMM_DTYPE),
            pltpu.VMEM((2, M_PER, N), COMM_DTYPE),
            pltpu.SemaphoreType.DMA((2,)),
            pltpu.SemaphoreType.DMA((2,)),
            pltpu.SemaphoreType.REGULAR,
        ],
        compiler_params=pltpu.CompilerParams(collective_id=0),
    )(x, w_mat, scale_x, scale_w)


# device time: 453601 ns/iter; 1.7633x vs baseline; 1.7633x over previous
import numpy as np

import jax
import jax.numpy as jnp
from jax import lax
from jax.experimental import pallas as pl
from jax.experimental.pallas import tpu as pltpu

P = 32
M_PER = 128
N = 8192
NH = N // 2
K_SH = 128
COMM_DTYPE = jnp.bfloat16

RING = [0, 1, 2, 3, 4, 5, 6, 7, 15, 14, 13, 12, 11, 10, 18, 19,
        20, 21, 22, 23, 31, 30, 29, 28, 27, 26, 25, 24, 16, 17, 9, 8]
SLOT = np.argsort(RING)


def kernel(x, w_mat, scale_x, scale_w):
    m_total, k_sh = x.shape
    _, n = w_mat.shape
    assert (m_total, k_sh, n) == (P * M_PER, K_SH, N), (x.shape, w_mat.shape)

    ring = jnp.asarray(RING, jnp.int32)
    slot = jnp.asarray(SLOT, jnp.int32)
    d = lax.axis_index("i")
    s = slot[d]
    hs = jnp.arange(P - 1, dtype=jnp.int32)
    sched_f = ring[(s - 1 - hs) % P]
    sched_b = ring[(s + 1 + hs) % P]
    nbrs = jnp.stack([ring[(s + 1) % P], ring[(s - 1) % P]])

    def body(x_ref, w_ref, sx_ref, sw_ref, schedf_ref, schedb_ref, nbr_ref,
             out_ref, recv_f, send_f, recv_b, send_b,
             ssem_f, rsem_f, ssem_b, rsem_b, credit_f, credit_b):
        my = lax.axis_index("i")
        succ_f = nbr_ref[0]
        succ_b = nbr_ref[1]

        barrier = pltpu.get_barrier_semaphore()
        for nbr in (succ_f, succ_b):
            pl.semaphore_signal(barrier, inc=1, device_id=(nbr,),
                                device_id_type=pl.DeviceIdType.MESH)
        pl.semaphore_wait(barrier, 2)

        def partial_chunk(c, lo):
            xa = x_ref[pl.ds(c * M_PER, M_PER), :]
            return lax.dot_general(
                xa, w_ref[:, lo:lo + NH], (((1,), (0,)), ((), ())),
                preferred_element_type=jnp.float32)

        def hop_rdma(h, sbuf, rbuf, ssem, rsem, tgt):
            sl = h % 2
            return pltpu.make_async_remote_copy(
                src_ref=sbuf.at[sl], dst_ref=rbuf.at[sl],
                send_sem=ssem.at[sl], recv_sem=rsem.at[sl],
                device_id=(tgt,), device_id_type=pl.DeviceIdType.MESH)

        rdma_f = lambda h: hop_rdma(h, send_f, recv_f, ssem_f, rsem_f, succ_f)
        rdma_b = lambda h: hop_rdma(h, send_b, recv_b, ssem_b, rsem_b, succ_b)

        for h in range(P - 1):
            pf = partial_chunk(schedf_ref[h], 0)
            pb = partial_chunk(schedb_ref[h], NH)
            if h > 0:
                rdma_f(h - 1).wait_recv()
                pf = pf + recv_f[(h - 1) % 2].astype(jnp.float32)
                rdma_b(h - 1).wait_recv()
                pb = pb + recv_b[(h - 1) % 2].astype(jnp.float32)
                if h - 1 <= P - 4:
                    pl.semaphore_signal(credit_f, inc=1, device_id=(succ_b,),
                                        device_id_type=pl.DeviceIdType.MESH)
                    pl.semaphore_signal(credit_b, inc=1, device_id=(succ_f,),
                                        device_id_type=pl.DeviceIdType.MESH)
            send_f[h % 2] = pf.astype(COMM_DTYPE)
            send_b[h % 2] = pb.astype(COMM_DTYPE)
            if h >= 2:
                pl.semaphore_wait(credit_f, 1)
                pl.semaphore_wait(credit_b, 1)
            rf = rdma_f(h)
            rb = rdma_b(h)
            rf.start()
            rb.start()
            rf.wait_send()
            rb.wait_send()

        rdma_f(P - 2).wait_recv()
        rdma_b(P - 2).wait_recv()
        acc_f = partial_chunk(my, 0) + recv_f[(P - 2) % 2].astype(jnp.float32)
        acc_b = partial_chunk(my, NH) + recv_b[(P - 2) % 2].astype(jnp.float32)
        scale = sx_ref[0] * sw_ref[0]
        out_ref[:, 0:NH] = jnp.maximum(acc_f * scale, 0.0)
        out_ref[:, NH:N] = jnp.maximum(acc_b * scale, 0.0)

    return pl.pallas_call(
        body,
        out_shape=jax.ShapeDtypeStruct((M_PER, N), jnp.float32),
        in_specs=[
            pl.BlockSpec(memory_space=pltpu.VMEM),
            pl.BlockSpec(memory_space=pltpu.VMEM),
            pl.BlockSpec(memory_space=pltpu.SMEM),
            pl.BlockSpec(memory_space=pltpu.SMEM),
            pl.BlockSpec(memory_space=pltpu.SMEM),
            pl.BlockSpec(memory_space=pltpu.SMEM),
            pl.BlockSpec(memory_space=pltpu.SMEM),
        ],
        out_specs=pl.BlockSpec(memory_space=pltpu.VMEM),
        scratch_shapes=[
            pltpu.VMEM((2, M_PER, NH), COMM_DTYPE),
            pltpu.VMEM((2, M_PER, NH), COMM_DTYPE),
            pltpu.VMEM((2, M_PER, NH), COMM_DTYPE),
            pltpu.VMEM((2, M_PER, NH), COMM_DTYPE),
            pltpu.SemaphoreType.DMA((2,)),
            pltpu.SemaphoreType.DMA((2,)),
            pltpu.SemaphoreType.DMA((2,)),
            pltpu.SemaphoreType.DMA((2,)),
            pltpu.SemaphoreType.REGULAR,
            pltpu.SemaphoreType.REGULAR,
        ],
        compiler_params=pltpu.CompilerParams(collective_id=0),
    )(x, w_mat, scale_x, scale_w, sched_f, sched_b, nbrs)


# device time: 367206 ns/iter; 2.1782x vs baseline; 1.2353x over previous
import numpy as np

import jax
import jax.numpy as jnp
from jax import lax
from jax.experimental import pallas as pl
from jax.experimental.pallas import tpu as pltpu

P = 32
M_PER = 128
N = 8192
NH = N // 2
NQ = NH // 2
K_SH = 128
COMM_DTYPE = jnp.bfloat16

RING = [0, 1, 2, 3, 4, 5, 6, 7, 15, 14, 13, 12, 11, 10, 18, 19,
        20, 21, 22, 23, 31, 30, 29, 28, 27, 26, 25, 24, 16, 17, 9, 8]
SLOT = np.argsort(RING)


def kernel(x, w_mat, scale_x, scale_w):
    m_total, k_sh = x.shape
    _, n = w_mat.shape
    assert (m_total, k_sh, n) == (P * M_PER, K_SH, N), (x.shape, w_mat.shape)

    ring = jnp.asarray(RING, jnp.int32)
    slot = jnp.asarray(SLOT, jnp.int32)
    d = lax.axis_index("i")
    s = slot[d]
    hs = jnp.arange(P - 1, dtype=jnp.int32)
    sched_f = ring[(s - 1 - hs) % P]
    sched_b = ring[(s + 1 + hs) % P]
    nbrs = jnp.stack([ring[(s + 1) % P], ring[(s - 1) % P]])

    def body(x_ref, w_ref, sx_ref, sw_ref, schedf_ref, schedb_ref, nbr_ref,
             out_ref, recv_f, send_f, recv_b, send_b,
             ssem_f, rsem_f, ssem_b, rsem_b, credit_f, credit_b):
        my = lax.axis_index("i")
        succ_f = nbr_ref[0]
        succ_b = nbr_ref[1]

        barrier = pltpu.get_barrier_semaphore()
        for nbr in (succ_f, succ_b):
            pl.semaphore_signal(barrier, inc=1, device_id=(nbr,),
                                device_id_type=pl.DeviceIdType.MESH)
        pl.semaphore_wait(barrier, 2)

        def partial_chunk(c, lo):
            xa = x_ref[pl.ds(c * M_PER, M_PER), :]
            return lax.dot_general(
                xa, w_ref[:, lo:lo + NH], (((1,), (0,)), ((), ())),
                preferred_element_type=jnp.float32)

        def hop_rdma(h, j, sbuf, rbuf, ssem, rsem, tgt):
            sl = h % 2
            return pltpu.make_async_remote_copy(
                src_ref=sbuf.at[sl, j], dst_ref=rbuf.at[sl, j],
                send_sem=ssem.at[sl, j], recv_sem=rsem.at[sl, j],
                device_id=(tgt,), device_id_type=pl.DeviceIdType.MESH)

        rdma = (
            lambda h, j: hop_rdma(h, j, send_f, recv_f, ssem_f, rsem_f, succ_f),
            lambda h, j: hop_rdma(h, j, send_b, recv_b, ssem_b, rsem_b, succ_b),
        )
        recv = (recv_f, recv_b)
        send = (send_f, send_b)
        credit = (credit_f, credit_b)
        upstream = (succ_b, succ_f)

        for h in range(P - 1):
            parts = [partial_chunk(schedf_ref[h], 0),
                     partial_chunk(schedb_ref[h], NH)]
            for j in range(2):
                for g in range(2):
                    p = parts[g][:, j * NQ:(j + 1) * NQ]
                    if h > 0:
                        rdma[g](h - 1, j).wait_recv()
                        p = p + recv[g][(h - 1) % 2, j].astype(jnp.float32)
                    if h >= 2:
                        rdma[g](h - 2, j).wait_send()
                    send[g][h % 2, j] = p.astype(COMM_DTYPE)
                    if h > 0 and j == 1 and h - 1 <= P - 4:
                        pl.semaphore_signal(
                            credit[g], inc=1, device_id=(upstream[g],),
                            device_id_type=pl.DeviceIdType.MESH)
                    if h >= 2 and j == 0:
                        pl.semaphore_wait(credit[g], 1)
                    rdma[g](h, j).start()

        scale = sx_ref[0] * sw_ref[0]
        accs = [partial_chunk(my, 0), partial_chunk(my, NH)]
        for g in range(2):
            for j in range(2):
                rdma[g](P - 2, j).wait_recv()
            fin = jnp.concatenate(
                [recv[g][(P - 2) % 2, 0], recv[g][(P - 2) % 2, 1]], axis=1)
            acc = accs[g] + fin.astype(jnp.float32)
            lo = g * NH
            out_ref[:, lo:lo + NH] = jnp.maximum(acc * scale, 0.0)
            for hh in (P - 3, P - 2):
                for j in range(2):
                    rdma[g](hh, j).wait_send()

    return pl.pallas_call(
        body,
        out_shape=jax.ShapeDtypeStruct((M_PER, N), jnp.float32),
        in_specs=[
            pl.BlockSpec(memory_space=pltpu.VMEM),
            pl.BlockSpec(memory_space=pltpu.VMEM),
            pl.BlockSpec(memory_space=pltpu.SMEM),
            pl.BlockSpec(memory_space=pltpu.SMEM),
            pl.BlockSpec(memory_space=pltpu.SMEM),
            pl.BlockSpec(memory_space=pltpu.SMEM),
            pl.BlockSpec(memory_space=pltpu.SMEM),
        ],
        out_specs=pl.BlockSpec(memory_space=pltpu.VMEM),
        scratch_shapes=[
            pltpu.VMEM((2, 2, M_PER, NQ), COMM_DTYPE),
            pltpu.VMEM((2, 2, M_PER, NQ), COMM_DTYPE),
            pltpu.VMEM((2, 2, M_PER, NQ), COMM_DTYPE),
            pltpu.VMEM((2, 2, M_PER, NQ), COMM_DTYPE),
            pltpu.SemaphoreType.DMA((2, 2)),
            pltpu.SemaphoreType.DMA((2, 2)),
            pltpu.SemaphoreType.DMA((2, 2)),
            pltpu.SemaphoreType.DMA((2, 2)),
            pltpu.SemaphoreType.REGULAR,
            pltpu.SemaphoreType.REGULAR,
        ],
        compiler_params=pltpu.CompilerParams(collective_id=0),
    )(x, w_mat, scale_x, scale_w, sched_f, sched_b, nbrs)


# device time: 266009 ns/iter; 3.0068x vs baseline; 1.3804x over previous
import numpy as np

import jax
import jax.numpy as jnp
from jax import lax
from jax.experimental import pallas as pl
from jax.experimental.pallas import tpu as pltpu

P = 32
M_PER = 128
N = 8192
K_SH = 128
COMM_DTYPE = jnp.bfloat16

RINGS = [
    [0, 1, 9, 8, 16, 17, 25, 24, 27, 26, 29, 28, 31, 30, 22, 23,
     20, 19, 18, 21, 13, 10, 2, 5, 6, 14, 15, 7, 4, 12, 11, 3],
    [0, 8, 11, 10, 9, 17, 18, 19, 16, 24, 25, 26, 27, 28, 20, 21,
     29, 30, 31, 23, 22, 14, 6, 7, 15, 12, 13, 5, 4, 3, 2, 1],
    [0, 3, 4, 7, 6, 5, 13, 14, 22, 30, 29, 21, 18, 26, 25, 17,
     9, 1, 2, 10, 11, 19, 20, 12, 15, 23, 31, 28, 27, 24, 16, 8],
]
NR = 3
WIDTHS = (2816, 2816, 2560)
LOS = (0, 2816, 5632)
SUBW = tuple(w // 2 for w in WIDTHS)


def kernel(x, w_mat, scale_x, scale_w):
    m_total, k_sh = x.shape
    _, n = w_mat.shape
    assert (m_total, k_sh, n) == (P * M_PER, K_SH, N), (x.shape, w_mat.shape)

    d = lax.axis_index("i")
    hs = jnp.arange(P - 1, dtype=jnp.int32)
    scheds = []
    succs = []
    preds = []
    for g in range(NR):
        ring = jnp.asarray(RINGS[g], jnp.int32)
        slot = jnp.asarray(np.argsort(RINGS[g]), jnp.int32)
        s = slot[d]
        scheds.append(ring[(s - 1 - hs) % P])
        succs.append(ring[(s + 1) % P])
        preds.append(ring[(s - 1) % P])
    nbrs = jnp.stack(succs + preds)

    def body(x_ref, w_ref, sx_ref, sw_ref, sched0_ref, sched1_ref, sched2_ref,
             nbr_ref, out_ref,
             recv0, send0, recv1, send1, recv2, send2,
             ssem0, rsem0, ssem1, rsem1, ssem2, rsem2,
             credit0, credit1, credit2):
        my = lax.axis_index("i")
        sched = (sched0_ref, sched1_ref, sched2_ref)
        recv = (recv0, recv1, recv2)
        send = (send0, send1, send2)
        ssem = (ssem0, ssem1, ssem2)
        rsem = (rsem0, rsem1, rsem2)
        credit = (credit0, credit1, credit2)
        succ = tuple(nbr_ref[g] for g in range(NR))
        pred = tuple(nbr_ref[NR + g] for g in range(NR))

        barrier = pltpu.get_barrier_semaphore()
        for nbr in succ + pred:
            pl.semaphore_signal(barrier, inc=1, device_id=(nbr,),
                                device_id_type=pl.DeviceIdType.MESH)
        pl.semaphore_wait(barrier, 2 * NR)

        def partial_chunk(c, g):
            xa = x_ref[pl.ds(c * M_PER, M_PER), :]
            return lax.dot_general(
                xa, w_ref[:, LOS[g]:LOS[g] + WIDTHS[g]],
                (((1,), (0,)), ((), ())),
                preferred_element_type=jnp.float32)

        def rdma(g, h, j):
            sl = h % 2
            return pltpu.make_async_remote_copy(
                src_ref=send[g].at[sl, j], dst_ref=recv[g].at[sl, j],
                send_sem=ssem[g].at[sl, j], recv_sem=rsem[g].at[sl, j],
                device_id=(succ[g],), device_id_type=pl.DeviceIdType.MESH)

        for h in range(P - 1):
            parts = [partial_chunk(sched[g][h], g) for g in range(NR)]
            for j in range(2):
                for g in range(NR):
                    w = SUBW[g]
                    p = parts[g][:, j * w:(j + 1) * w]
                    if h > 0:
                        rdma(g, h - 1, j).wait_recv()
                        p = p + recv[g][(h - 1) % 2, j].astype(jnp.float32)
                    if h >= 2:
                        rdma(g, h - 2, j).wait_send()
                    send[g][h % 2, j] = p.astype(COMM_DTYPE)
                    if h > 0 and j == 1 and h - 1 <= P - 4:
                        pl.semaphore_signal(
                            credit[g], inc=1, device_id=(pred[g],),
                            device_id_type=pl.DeviceIdType.MESH)
                    if h >= 2 and j == 0:
                        pl.semaphore_wait(credit[g], 1)
                    rdma(g, h, j).start()

        scale = sx_ref[0] * sw_ref[0]
        for g in range(NR):
            for j in range(2):
                rdma(g, P - 2, j).wait_recv()
            fin = jnp.concatenate(
                [recv[g][(P - 2) % 2, 0], recv[g][(P - 2) % 2, 1]], axis=1)
            acc = partial_chunk(my, g) + fin.astype(jnp.float32)
            lo = LOS[g]
            out_ref[:, lo:lo + WIDTHS[g]] = jnp.maximum(acc * scale, 0.0)
            for hh in (P - 3, P - 2):
                for j in range(2):
                    rdma(g, hh, j).wait_send()

    return pl.pallas_call(
        body,
        out_shape=jax.ShapeDtypeStruct((M_PER, N), jnp.float32),
        in_specs=[
            pl.BlockSpec(memory_space=pltpu.VMEM),
            pl.BlockSpec(memory_space=pltpu.VMEM),
            pl.BlockSpec(memory_space=pltpu.SMEM),
            pl.BlockSpec(memory_space=pltpu.SMEM),
            pl.BlockSpec(memory_space=pltpu.SMEM),
            pl.BlockSpec(memory_space=pltpu.SMEM),
            pl.BlockSpec(memory_space=pltpu.SMEM),
            pl.BlockSpec(memory_space=pltpu.SMEM),
        ],
        out_specs=pl.BlockSpec(memory_space=pltpu.VMEM),
        scratch_shapes=[
            pltpu.VMEM((2, 2, M_PER, SUBW[0]), COMM_DTYPE),
            pltpu.VMEM((2, 2, M_PER, SUBW[0]), COMM_DTYPE),
            pltpu.VMEM((2, 2, M_PER, SUBW[1]), COMM_DTYPE),
            pltpu.VMEM((2, 2, M_PER, SUBW[1]), COMM_DTYPE),
            pltpu.VMEM((2, 2, M_PER, SUBW[2]), COMM_DTYPE),
            pltpu.VMEM((2, 2, M_PER, SUBW[2]), COMM_DTYPE),
            pltpu.SemaphoreType.DMA((2, 2)),
            pltpu.SemaphoreType.DMA((2, 2)),
            pltpu.SemaphoreType.DMA((2, 2)),
            pltpu.SemaphoreType.DMA((2, 2)),
            pltpu.SemaphoreType.DMA((2, 2)),
            pltpu.SemaphoreType.DMA((2, 2)),
            pltpu.SemaphoreType.REGULAR,
            pltpu.SemaphoreType.REGULAR,
            pltpu.SemaphoreType.REGULAR,
        ],
        compiler_params=pltpu.CompilerParams(collective_id=0),
    )(x, w_mat, scale_x, scale_w, scheds[0], scheds[1], scheds[2], nbrs)
